# baseline (speedup 1.0000x reference)
import functools

import jax
import jax.numpy as jnp
from jax import lax
from jax.experimental import pallas as pl
from jax.experimental.pallas import tpu as pltpu
from jax.experimental.pallas import tpu_sc as plsc

K = 8
ROWS = 64
N = 1_000_000
L = 16
BLK = 2000
NBLK = N // BLK
NBLK_PAD = 512
CHUNK = 40_000
NCHUNK = N // CHUNK
BLK_PER_CHUNK = CHUNK // BLK
VPB = BLK // L
CAND_CAP = 4096
NEG_INF = float("-inf")
BIG_I = 2**30
NC = 2
NWORKERS = 32
ROWS_PER_W = ROWS // NWORKERS


def _body(x_ref, vals_ref, idxs_ref,
          buf, blkbuf, bmv, candv, candi, ov, oi, bms, sem_a, sem_b):
    cid = lax.axis_index("c")
    sid = lax.axis_index("s")
    wid = sid * NC + cid
    iota = lax.broadcasted_iota(jnp.int32, (L,), 0)
    ninf = jnp.full((L,), NEG_INF, jnp.float32)

    for r in range(ROWS_PER_W):
        row = wid * ROWS_PER_W + r
        base = row * N

        def fill_bmv(i, _):
            bmv[pl.ds(i * L, L)] = ninf
            return 0
        lax.fori_loop(0, NBLK_PAD // L, fill_bmv, 0)

        sems = (sem_a, sem_b)

        def dma_chunk(c):
            return pltpu.make_async_copy(
                x_ref.at[pl.ds(base + c * CHUNK, CHUNK)],
                buf.at[pl.ds((c % 2) * CHUNK, CHUNK)],
                sems[c % 2])

        dma_chunk(0).start()
        for c in range(NCHUNK):
            if c + 1 < NCHUNK:
                dma_chunk(c + 1).start()
            dma_chunk(c).wait()
            coff = (c % 2) * CHUNK

            def blk_body(blk, _, coff=coff, c=c):
                def inner(i, acc):
                    return jnp.maximum(
                        acc, buf[pl.ds(coff + blk * BLK + i * L, L)])
                acc = lax.fori_loop(0, VPB, inner, ninf)
                m = jnp.max(acc)
                g = c * BLK_PER_CHUNK + blk
                bms[g] = m
                gb = (g // L) * L
                lane = g % L
                v = bmv[pl.ds(gb, L)]
                bmv[pl.ds(gb, L)] = jnp.where(iota == lane, m, v)
                return 0
            lax.fori_loop(0, BLK_PER_CHUNK, blk_body, 0)

        t = jnp.float32(0)
        for _ in range(K):
            def maxb(i, acc):
                return jnp.maximum(acc, bmv[pl.ds(i * L, L)])
            m = jnp.max(lax.fori_loop(0, NBLK_PAD // L, maxb, ninf))

            def rm(i, done):
                v = bmv[pl.ds(i * L, L)]
                eq = v == m
                pc = plsc.all_reduce_population_count(eq)
                ffs = plsc.all_reduce_ffs(eq)
                has = pc > 0
                do = jnp.logical_and(has, done == 0)
                hit = jnp.logical_and(do, iota == ffs)
                bmv[pl.ds(i * L, L)] = jnp.where(hit, NEG_INF, v)
                return jnp.where(has, jnp.ones_like(done), done)
            lax.fori_loop(0, NBLK_PAD // L, rm, jnp.zeros((L,), jnp.int32))
            t = m

        def fill_cand(i, _):
            candv[pl.ds(i * L, L)] = ninf
            candi[pl.ds(i * L, L)] = jnp.full((L,), -1, jnp.int32)
            return 0
        lax.fori_loop(0, CAND_CAP // L, fill_cand, 0)

        def scan_blk(b, cnt):
            def do_scan(cnt):
                pltpu.sync_copy(x_ref.at[pl.ds(base + b * BLK, BLK)], blkbuf)

                def inner(i, cnt):
                    v = blkbuf[pl.ds(i * L, L)]
                    mask = v >= t
                    hasc = jnp.max(
                        jnp.where(mask, jnp.int32(1), jnp.int32(0))) > 0
                    ok = jnp.logical_and(hasc, cnt <= CAND_CAP - L)

                    def app(cnt):
                        gidx = b * BLK + i * L + iota
                        candv[pl.ds(cnt, L)] = jnp.where(mask, v, ninf)
                        candi[pl.ds(cnt, L)] = jnp.where(mask, gidx, -1)
                        return cnt + L
                    return lax.cond(ok, app, lambda c_: c_, cnt)
                return lax.fori_loop(0, VPB, inner, cnt)
            return lax.cond(bms[b] >= t, do_scan, lambda c_: c_, cnt)
        cnt = lax.fori_loop(0, NBLK, scan_blk, jnp.int32(0))
        nb = cnt // L

        wvals = ninf
        widxs = jnp.full((L,), -1, jnp.int32)
        for rr in range(K):
            def am(i, carry):
                accv, acci = carry
                v = candv[pl.ds(i * L, L)]
                idv = candi[pl.ds(i * L, L)]
                gt = v > accv
                eq = v == accv
                take = jnp.logical_or(
                    gt, jnp.logical_and(eq, idv < acci))
                return (jnp.where(take, v, accv),
                        jnp.where(take, idv, acci))
            accv, acci = lax.fori_loop(
                0, nb, am, (ninf, jnp.full((L,), BIG_I, jnp.int32)))
            m = jnp.max(accv)
            mi = jnp.min(jnp.where(accv == m, acci, BIG_I))
            wvals = jnp.where(iota == rr, m, wvals)
            widxs = jnp.where(iota == rr, mi, widxs)

            def rmw(i, _):
                v = candv[pl.ds(i * L, L)]
                idv = candi[pl.ds(i * L, L)]
                candv[pl.ds(i * L, L)] = jnp.where(idv == mi, NEG_INF, v)
                return 0
            lax.fori_loop(0, nb, rmw, 0)

        ov[...] = wvals
        oi[...] = widxs
        pltpu.sync_copy(ov, vals_ref.at[pl.ds(row * L, L)])
        pltpu.sync_copy(oi, idxs_ref.at[pl.ds(row * L, L)])


_mesh = plsc.VectorSubcoreMesh(core_axis_name="c", subcore_axis_name="s")

_sc_call = functools.partial(
    pl.kernel,
    mesh=_mesh,
    compiler_params=pltpu.CompilerParams(needs_layout_passes=False),
    out_type=[
        jax.ShapeDtypeStruct((ROWS * L,), jnp.float32),
        jax.ShapeDtypeStruct((ROWS * L,), jnp.int32),
    ],
    scratch_types=[
        pltpu.VMEM((2 * CHUNK,), jnp.float32),
        pltpu.VMEM((BLK,), jnp.float32),
        pltpu.VMEM((NBLK_PAD,), jnp.float32),
        pltpu.VMEM((CAND_CAP,), jnp.float32),
        pltpu.VMEM((CAND_CAP,), jnp.int32),
        pltpu.VMEM((L,), jnp.float32),
        pltpu.VMEM((L,), jnp.int32),
        pltpu.SMEM((NBLK,), jnp.float32),
        pltpu.SemaphoreType.DMA,
        pltpu.SemaphoreType.DMA,
    ],
)(_body)


@jax.jit
def kernel(x):
    vals_flat, idxs_flat = _sc_call(x.reshape(-1))
    vals = vals_flat.reshape(ROWS, L)[:, :K]
    idxs = idxs_flat.reshape(ROWS, L)[:, :K]
    return vals, idxs
